# Initial kernel scaffold; baseline (speedup 1.0000x reference)
import jax
import jax.numpy as jnp
from jax.experimental import pallas as pl


def kernel(index, emb, height_emb, width_emb):
    raise NotImplementedError("write your pallas kernel here")


# speedup vs baseline: 1.5614x; 1.5614x over previous
import functools

import jax
import jax.numpy as jnp
from jax import lax
from jax.experimental import pallas as pl
from jax.experimental.pallas import tpu as pltpu
from jax.experimental.pallas import tpu_sc as plsc

_NUM_EMBED = 8192
_HEIGHT = 32
_WIDTH = 32
_D = 512
_B = 128
_HW = _HEIGHT * _WIDTH
_ROWS = _B * _HW

_NC = 2
_NS = 16
_NW = _NC * _NS
_RPW = _ROWS // _NW
_CH = 64
_NCH = _RPW // _CH
_PSTAGE = _HW // _NS


def _pos_body(h_ref, w_ref, o_ref):
    o_ref[...] = h_ref[...][:, None, :] + w_ref[...][None, :, :]


def _build_pos(height_emb, width_emb):
    pos = pl.pallas_call(
        _pos_body,
        out_shape=jax.ShapeDtypeStruct((_HEIGHT, _WIDTH, _D), jnp.float32),
    )(height_emb, width_emb)
    return pos.reshape(_HW, _D)


def _sc_body(idx_hbm, emb_hbm, pos_hbm, out_hbm, idx_v, buf, pbuf, pos_sh, sem):
    c = lax.axis_index("c")
    s = lax.axis_index("s")
    wid = s * _NC + c
    pltpu.sync_copy(pos_hbm.at[pl.ds(s * _PSTAGE, _PSTAGE)],
                    pos_sh.at[pl.ds(s * _PSTAGE, _PSTAGE)])
    base = wid * _RPW
    pltpu.sync_copy(idx_hbm.at[pl.ds(base, _RPW)], idx_v)
    plsc.subcore_barrier()

    def chunk(ci, carry):
        r0 = ci * _CH
        p0 = lax.rem(r0, _HW)
        pltpu.sync_copy(pos_sh.at[pl.ds(p0, _CH)], pbuf)
        pltpu.async_copy(emb_hbm.at[idx_v.at[pl.ds(r0, _CH)]], buf, sem).wait()

        def addrow(r, c2):
            for j in range(_D // 16):
                sl = pl.ds(j * 16, 16)
                buf[r, sl] = buf[r, sl] + pbuf[r, sl]
            return c2

        lax.fori_loop(0, _CH, addrow, 0)
        pltpu.sync_copy(buf, out_hbm.at[pl.ds(base + r0, _CH)])
        return carry

    lax.fori_loop(0, _NCH, chunk, 0)


@functools.partial(jax.jit, static_argnames=())
def _lookup(idx_flat, emb, pos):
    mesh = plsc.VectorSubcoreMesh(core_axis_name="c", subcore_axis_name="s")
    f = pl.kernel(
        _sc_body,
        out_type=jax.ShapeDtypeStruct((_ROWS, _D), jnp.float32),
        mesh=mesh,
        scratch_types=[
            pltpu.VMEM((_RPW,), jnp.int32),
            pltpu.VMEM((_CH, _D), jnp.float32),
            pltpu.VMEM((_CH, _D), jnp.float32),
            pltpu.VMEM_SHARED((_HW, _D), jnp.float32),
            pltpu.SemaphoreType.DMA,
        ],
    )
    return f(idx_flat, emb, pos)


def kernel(index, emb, height_emb, width_emb):
    pos = _build_pos(height_emb, width_emb)
    out = _lookup(index.reshape(_ROWS), emb, pos)
    return out.reshape(_B, _HW, _D)
